# baseline (speedup 1.0000x reference)
import jax
import jax.numpy as jnp
from jax.experimental import pallas as pl

_EPS = 1e-5


def _ln_kernel(x_ref, e_ref, g_ref, b_ref, o_ref):
    g = g_ref[...]
    b = b_ref[...]
    rows = x_ref.shape[0]
    cr = 128
    for k in range(rows // cr):
        sl = pl.ds(k * cr, cr)
        h = x_ref[sl, :] + e_ref[sl, :]
        mean = jnp.mean(h, axis=-1, keepdims=True)
        c = h - mean
        var = jnp.mean(c * c, axis=-1, keepdims=True)
        o_ref[sl, :] = c * jax.lax.rsqrt(var + _EPS) * g + b


def kernel(x, emb, gamma, beta):
    S, H = x.shape
    BR = 512
    g2 = gamma.reshape(1, H)
    b2 = beta.reshape(1, H)
    return pl.pallas_call(
        _ln_kernel,
        grid=(S // BR,),
        in_specs=[
            pl.BlockSpec((BR, H), lambda i: (i, 0)),
            pl.BlockSpec((BR, H), lambda i: (i, 0)),
            pl.BlockSpec((1, H), lambda i: (0, 0)),
            pl.BlockSpec((1, H), lambda i: (0, 0)),
        ],
        out_specs=pl.BlockSpec((BR, H), lambda i: (i, 0)),
        out_shape=jax.ShapeDtypeStruct((S, H), x.dtype),
    )(x, emb, g2, b2)


# speedup vs baseline: 1.0014x; 1.0014x over previous
import jax
import jax.numpy as jnp
from jax.experimental import pallas as pl

_EPS = 1e-5


def _ln_kernel(x_ref, e_ref, g_ref, b_ref, o_ref):
    g = g_ref[...]
    b = b_ref[...]
    rows = x_ref.shape[0]
    cr = 128
    for k in range(rows // cr):
        sl = pl.ds(k * cr, cr)
        h = x_ref[sl, :] + e_ref[sl, :]
        inv_h = 1.0 / h.shape[-1]
        mean = jnp.sum(h, axis=-1, keepdims=True) * inv_h
        var = jnp.sum(h * h, axis=-1, keepdims=True) * inv_h - mean * mean
        scale = jax.lax.rsqrt(var + _EPS) * g
        o_ref[sl, :] = (h - mean) * scale + b


def kernel(x, emb, gamma, beta):
    S, H = x.shape
    BR = 512
    g2 = gamma.reshape(1, H)
    b2 = beta.reshape(1, H)
    return pl.pallas_call(
        _ln_kernel,
        grid=(S // BR,),
        in_specs=[
            pl.BlockSpec((BR, H), lambda i: (i, 0)),
            pl.BlockSpec((BR, H), lambda i: (i, 0)),
            pl.BlockSpec((1, H), lambda i: (0, 0)),
            pl.BlockSpec((1, H), lambda i: (0, 0)),
        ],
        out_specs=pl.BlockSpec((BR, H), lambda i: (i, 0)),
        out_shape=jax.ShapeDtypeStruct((S, H), x.dtype),
    )(x, emb, g2, b2)
